# Initial kernel scaffold; baseline (speedup 1.0000x reference)
import jax
import jax.numpy as jnp
from jax.experimental import pallas as pl


def kernel(wav, offsets):
    raise NotImplementedError("write your pallas kernel here")


# speedup vs baseline: 2.6393x; 2.6393x over previous
import functools

import jax
import jax.numpy as jnp
from jax import lax
from jax.experimental import pallas as pl
from jax.experimental.pallas import tpu as pltpu
from jax.experimental.pallas import tpu_sc as plsc

SHIFT = 8192
B, S, C, T = 16, 4, 2, 441000
L = T - SHIFT
ROWS = B * S * C
NC, NS = 2, 16
NW = NC * NS
ROWS_PER_W = ROWS // NW

CHUNK = 32256
NFULL = L // CHUNK
REM = L - NFULL * CHUNK
NOFF = ROWS // C
UNROLL = 8
PAD = UNROLL * 16


def _make_kernel():
    mesh = plsc.VectorSubcoreMesh(core_axis_name="c", subcore_axis_name="s")

    @functools.partial(
        pl.kernel,
        out_type=jax.ShapeDtypeStruct((ROWS * L,), jnp.float32),
        mesh=mesh,
        scratch_types=[
            pltpu.VMEM((CHUNK + 8 + PAD,), jnp.float32),
            pltpu.VMEM((CHUNK + 8 + PAD,), jnp.float32),
            pltpu.VMEM((CHUNK + PAD,), jnp.float32),
            pltpu.VMEM((CHUNK + PAD,), jnp.float32),
            pltpu.VMEM((NOFF + 16,), jnp.int32),
            pltpu.SemaphoreType.DMA,
            pltpu.SemaphoreType.DMA,
        ],
    )
    def shift_kernel(wav_hbm, off_hbm, out_hbm, in0, in1, ob0, ob1, off_v,
                     sem_g, sem_s):
        ins = (in0, in1)
        obs = (ob0, ob1)
        wid = lax.axis_index("s") * NC + lax.axis_index("c")
        pltpu.sync_copy(off_hbm, off_v.at[pl.ds(0, NOFF)])

        steps = []
        for j in range(ROWS_PER_W):
            for i in range(NFULL):
                steps.append((j, i * CHUNK, CHUNK))
            if REM:
                steps.append((j, NFULL * CHUNK, REM))
        nsteps = len(steps)

        aligned = []
        for j in range(ROWS_PER_W):
            off = off_v[pl.ds(wid * ROWS_PER_W // C + j // C, 16)][0]
            a = lax.bitwise_and(off, -8)
            aligned.append((pl.multiple_of(a, 8), off - a))

        def gather(k):
            j, start, n = steps[k]
            row = wid * ROWS_PER_W + j
            a, _ = aligned[j]
            cp = pltpu.make_async_copy(
                wav_hbm.at[pl.ds(pl.multiple_of(row * T + a + start, 8), n + 8)],
                ins[k % 2].at[pl.ds(0, n + 8)],
                sem_g,
            )
            cp.start()
            return cp

        def scatter(k):
            j, start, n = steps[k]
            row = wid * ROWS_PER_W + j
            cp = pltpu.make_async_copy(
                obs[k % 2].at[pl.ds(0, n)],
                out_hbm.at[pl.ds(pl.multiple_of(row * L + start, 8), n)],
                sem_s,
            )
            cp.start()
            return cp

        def shift(k):
            j, _, n = steps[k]
            _, r = aligned[j]
            src, dst = ins[k % 2], obs[k % 2]
            n_groups = pl.cdiv(pl.cdiv(n, 16), UNROLL)

            def body(t, carry):
                base = t * (UNROLL * 16)
                for u in range(UNROLL):
                    p = base + u * 16
                    dst[pl.ds(p, 16)] = src[pl.ds(r + p, 16)]
                return carry

            lax.fori_loop(0, n_groups, body, 0, unroll=False)

        g = gather(0)
        scatters = [None, None]
        for k in range(nsteps):
            g.wait()
            if k + 1 < nsteps:
                g = gather(k + 1)
            if scatters[k % 2] is not None:
                scatters[k % 2].wait()
            shift(k)
            scatters[k % 2] = scatter(k)
        scatters[(nsteps - 1) % 2].wait()
        if nsteps >= 2:
            scatters[(nsteps - 2) % 2].wait()

    return shift_kernel


_shift_kernel = _make_kernel()


def kernel(wav, offsets):
    wav_flat = wav.reshape(ROWS * T)
    offs = offsets.reshape(NOFF).astype(jnp.int32)
    out_flat = _shift_kernel(wav_flat, offs)
    return out_flat.reshape(B, S, C, L)
